# baseline (speedup 1.0000x reference)
import functools

import numpy as np
import jax
import jax.numpy as jnp
from jax import lax
from jax.experimental import pallas as pl
from jax.experimental.pallas import tpu as pltpu
from jax.experimental.pallas import tpu_sc as plsc

DIM = 64
_NC = 2
_NS = 16
_NW = _NC * _NS
_BB = 128
_PAD = 1


def _position_encoding(seq_len, d_model):
    positions = np.arange(seq_len)[:, np.newaxis]
    dims = np.arange(d_model)[np.newaxis, :]
    angles = positions / np.power(10000, 2 * (dims // 2) / d_model)
    pe = np.zeros(angles.shape, dtype=np.float32)
    pe[:, 0::2] = np.sin(angles[:, 0::2])
    pe[:, 1::2] = np.cos(angles[:, 1::2])
    return pe


@functools.lru_cache(maxsize=None)
def _make_kernel(B, L):
    NBB = B // _BB
    n_tasks = L * NBB
    assert B % _BB == 0 and n_tasks % (2 * _NW) == 0
    t_w = n_tasks // _NW
    per_w = t_w * _BB
    DH = DIM // 8
    ROW = _BB * 8

    mesh = plsc.VectorSubcoreMesh(core_axis_name="c", subcore_axis_name="s")

    @functools.partial(
        pl.kernel,
        mesh=mesh,
        compiler_params=pltpu.CompilerParams(
            use_tc_tiling_on_sc=False, needs_layout_passes=False,
            disable_bounds_checks=True),
        out_type=jax.ShapeDtypeStruct((L, DH, NBB, 8, _BB), jnp.float32),
        scratch_types=[
            pltpu.VMEM((per_w,), jnp.int32),
            pltpu.VMEM((L, DIM), jnp.float32),
            pltpu.VMEM((_BB, DIM), jnp.float32),
            pltpu.VMEM((_BB, DIM), jnp.float32),
            pltpu.VMEM((DH, 8, _BB + _PAD), jnp.float32),
            pltpu.VMEM((DH, 8, _BB + _PAD), jnp.float32),
            pltpu.SemaphoreType.DMA,
            pltpu.SemaphoreType.DMA,
            pltpu.SemaphoreType.DMA,
            pltpu.SemaphoreType.DMA,
        ],
    )
    def k(idx_hbm, table_hbm, pe_hbm, out_hbm,
          idx_v, pe_v, g0, g1, t0, t1, sg0, sg1, so0, so1):
        wid = lax.axis_index("s") * _NC + lax.axis_index("c")
        wbase = wid * per_w
        tbase = wid * t_w
        pltpu.sync_copy(idx_hbm.at[pl.ds(wbase, per_w)], idx_v)
        pltpu.sync_copy(pe_hbm, pe_v)

        gbuf = (g0, g1)
        tbuf = (t0, t1)
        gsem = (sg0, sg1)
        osem = (so0, so1)

        lane = lax.iota(jnp.int32, 16)
        dh_vecs = [(16 * g + lane) >> 3 for g in range(DIM // 16)]
        dlo_vecs = [(16 * g + lane) & 7 for g in range(DIM // 16)]

        def gather_start(i, b):
            pltpu.make_async_copy(
                table_hbm.at[idx_v.at[pl.ds(i * _BB, _BB)]], gbuf[b],
                gsem[b]).start()

        def gather_wait(b):
            pltpu.make_async_copy(
                table_hbm.at[idx_v.at[pl.ds(0, _BB)]], gbuf[b],
                gsem[b]).wait()

        def out_descs(i, b):
            t = tbase + i
            l = t // NBB
            bh = lax.rem(t, NBB)
            return pltpu.make_async_copy(
                tbuf[b].at[:, :, pl.ds(0, _BB)], out_hbm.at[l, :, bh],
                osem[b])

        def out_start(i, b):
            out_descs(i, b).start()

        def out_wait(i, b):
            out_descs(i, b).wait()

        gather_start(0, 0)
        gather_start(1, 1)

        def step(cc, carry):
            for b in range(2):
                i = 2 * cc + b
                t = tbase + i
                l = t // NBB
                gather_wait(b)

                @pl.when(i >= 2)
                def _():
                    out_wait(i, b)

                gb, tb = gbuf[b], tbuf[b]
                pe_rows = [pe_v[l, pl.ds(16 * g, 16)]
                           for g in range(DIM // 16)]

                @plsc.parallel_loop(0, _BB, unroll=8)
                def _(r):
                    b_vec = jnp.full((16,), r, jnp.int32)
                    for g in range(DIM // 16):
                        v = gb[r, pl.ds(16 * g, 16)] + pe_rows[g]
                        plsc.store_scatter(
                            tb, [dh_vecs[g], dlo_vecs[g], b_vec], v)

                out_start(i, b)

                @pl.when(i + 2 < t_w)
                def _():
                    gather_start(i + 2, b)
            return carry

        lax.fori_loop(0, t_w // 2, step, 0)

        out_wait(t_w - 2, 0)
        out_wait(t_w - 1, 1)

    return k


def kernel(inputs, table):
    B, L = inputs.shape
    pe = jnp.asarray(_position_encoding(L, DIM))
    idx_t = inputs.T.reshape(-1)
    out5 = _make_kernel(B, L)(idx_t, table, pe)
    return out5.transpose((2, 4, 0, 1, 3)).reshape(B, L, DIM)


# speedup vs baseline: 1.0667x; 1.0667x over previous
import functools

import numpy as np
import jax
import jax.numpy as jnp
from jax import lax
from jax.experimental import pallas as pl
from jax.experimental.pallas import tpu as pltpu
from jax.experimental.pallas import tpu_sc as plsc

DIM = 64
_NC = 2
_NS = 16
_NW = _NC * _NS
_BB = 128
_PAD = 1


def _position_encoding(seq_len, d_model):
    positions = np.arange(seq_len)[:, np.newaxis]
    dims = np.arange(d_model)[np.newaxis, :]
    angles = positions / np.power(10000, 2 * (dims // 2) / d_model)
    pe = np.zeros(angles.shape, dtype=np.float32)
    pe[:, 0::2] = np.sin(angles[:, 0::2])
    pe[:, 1::2] = np.cos(angles[:, 1::2])
    return pe


@functools.lru_cache(maxsize=None)
def _make_kernel(B, L):
    NBB = B // _BB
    n_tasks = L * NBB
    assert B % _BB == 0 and n_tasks % (2 * _NW) == 0
    t_w = n_tasks // _NW
    per_w = t_w * _BB
    DH = DIM // 8
    ROW = _BB * 8

    mesh = plsc.VectorSubcoreMesh(core_axis_name="c", subcore_axis_name="s")

    @functools.partial(
        pl.kernel,
        mesh=mesh,
        compiler_params=pltpu.CompilerParams(
            use_tc_tiling_on_sc=False, needs_layout_passes=False,
            disable_bounds_checks=True),
        out_type=jax.ShapeDtypeStruct((L, DH, NBB, 8, _BB), jnp.float32),
        scratch_types=[
            pltpu.VMEM((per_w,), jnp.int32),
            pltpu.VMEM((L, DIM), jnp.float32),
            pltpu.VMEM((_BB, DIM), jnp.float32),
            pltpu.VMEM((_BB, DIM), jnp.float32),
            pltpu.VMEM((DH, 8, _BB + _PAD), jnp.float32),
            pltpu.VMEM((DH, 8, _BB + _PAD), jnp.float32),
            pltpu.SemaphoreType.DMA,
            pltpu.SemaphoreType.DMA,
            pltpu.SemaphoreType.DMA,
            pltpu.SemaphoreType.DMA,
        ],
    )
    def k(idx_hbm, table_hbm, pe_hbm, out_hbm,
          idx_v, pe_v, g0, g1, t0, t1, sg0, sg1, so0, so1):
        wid = lax.axis_index("s") * _NC + lax.axis_index("c")
        wbase = wid * per_w
        tbase = wid * t_w
        pltpu.sync_copy(idx_hbm.at[pl.ds(wbase, per_w)], idx_v)
        pltpu.sync_copy(pe_hbm, pe_v)

        gbuf = (g0, g1)
        tbuf = (t0, t1)
        gsem = (sg0, sg1)
        osem = (so0, so1)

        lane = lax.iota(jnp.int32, 16)
        dh_vecs = [(16 * g + lane) >> 3 for g in range(DIM // 16)]
        dlo_vecs = [(16 * g + lane) & 7 for g in range(DIM // 16)]

        def gather_start(i, b):
            pltpu.make_async_copy(
                table_hbm.at[idx_v.at[pl.ds(i * _BB, _BB)]], gbuf[b],
                gsem[b]).start()

        def gather_wait(b):
            pltpu.make_async_copy(
                table_hbm.at[idx_v.at[pl.ds(0, _BB)]], gbuf[b],
                gsem[b]).wait()

        def out_descs(i, b):
            t = tbase + i
            l = t // NBB
            bh = lax.rem(t, NBB)
            return pltpu.make_async_copy(
                tbuf[b].at[:, :, pl.ds(0, _BB)], out_hbm.at[l, :, bh],
                osem[b])

        def out_start(i, b):
            out_descs(i, b).start()

        def out_wait(i, b):
            out_descs(i, b).wait()

        gather_start(0, 0)
        gather_start(1, 1)

        def step(cc, carry):
            for b in range(2):
                i = 2 * cc + b
                t = tbase + i
                l = t // NBB
                gather_wait(b)

                @pl.when(i >= 2)
                def _():
                    out_wait(i, b)

                gb, tb = gbuf[b], tbuf[b]
                pe_rows = [pe_v[l, pl.ds(16 * g, 16)]
                           for g in range(DIM // 16)]

                @plsc.parallel_loop(0, _BB, unroll=4)
                def _(r):
                    b_vec = jnp.full((16,), r, jnp.int32)
                    for g in range(DIM // 16):
                        v = gb[r, pl.ds(16 * g, 16)] + pe_rows[g]
                        plsc.store_scatter(
                            tb, [dh_vecs[g], dlo_vecs[g], b_vec], v)

                out_start(i, b)

                @pl.when(i + 2 < t_w)
                def _():
                    gather_start(i + 2, b)
            return carry

        lax.fori_loop(0, t_w // 2, step, 0)

        out_wait(t_w - 2, 0)
        out_wait(t_w - 1, 1)

    return k


def kernel(inputs, table):
    B, L = inputs.shape
    pe = jnp.asarray(_position_encoding(L, DIM))
    idx_t = inputs.T.reshape(-1)
    out5 = _make_kernel(B, L)(idx_t, table, pe)
    return out5.transpose((2, 4, 0, 1, 3)).reshape(B, L, DIM)


# speedup vs baseline: 1.0672x; 1.0005x over previous
import functools

import numpy as np
import jax
import jax.numpy as jnp
from jax import lax
from jax.experimental import pallas as pl
from jax.experimental.pallas import tpu as pltpu
from jax.experimental.pallas import tpu_sc as plsc

DIM = 64
_NC = 2
_NS = 16
_NW = _NC * _NS
_BB = 128
_PAD = 1


def _position_encoding(seq_len, d_model):
    positions = np.arange(seq_len)[:, np.newaxis]
    dims = np.arange(d_model)[np.newaxis, :]
    angles = positions / np.power(10000, 2 * (dims // 2) / d_model)
    pe = np.zeros(angles.shape, dtype=np.float32)
    pe[:, 0::2] = np.sin(angles[:, 0::2])
    pe[:, 1::2] = np.cos(angles[:, 1::2])
    return pe


@functools.lru_cache(maxsize=None)
def _make_kernel(B, L):
    NBB = B // _BB
    n_tasks = L * NBB
    assert B % _BB == 0 and n_tasks % (2 * _NW) == 0
    t_w = n_tasks // _NW
    per_w = t_w * _BB
    DH = DIM // 8
    ROW = _BB * 8

    mesh = plsc.VectorSubcoreMesh(core_axis_name="c", subcore_axis_name="s")

    @functools.partial(
        pl.kernel,
        mesh=mesh,
        compiler_params=pltpu.CompilerParams(
            use_tc_tiling_on_sc=False, needs_layout_passes=False,
            disable_bounds_checks=True),
        out_type=jax.ShapeDtypeStruct((L, DH, NBB, 8, _BB), jnp.float32),
        scratch_types=[
            pltpu.VMEM((per_w,), jnp.int32),
            pltpu.VMEM((L, DIM), jnp.float32),
            pltpu.VMEM((_BB, DIM), jnp.float32),
            pltpu.VMEM((_BB, DIM), jnp.float32),
            pltpu.VMEM((DH, 8, _BB + _PAD), jnp.float32),
            pltpu.VMEM((DH, 8, _BB + _PAD), jnp.float32),
            pltpu.SemaphoreType.DMA,
            pltpu.SemaphoreType.DMA,
            pltpu.SemaphoreType.DMA,
            pltpu.SemaphoreType.DMA,
        ],
    )
    def k(idx_hbm, table_hbm, pe_hbm, out_hbm,
          idx_v, pe_v, g0, g1, t0, t1, sg0, sg1, so0, so1):
        wid = lax.axis_index("s") * _NC + lax.axis_index("c")
        wbase = wid * per_w
        tbase = wid * t_w
        pltpu.sync_copy(idx_hbm.at[pl.ds(wbase, per_w)], idx_v)
        pltpu.sync_copy(pe_hbm, pe_v)

        gbuf = (g0, g1)
        tbuf = (t0, t1)
        gsem = (sg0, sg1)
        osem = (so0, so1)

        lane = lax.iota(jnp.int32, 16)
        dh_vecs = [(16 * g + lane) >> 3 for g in range(DIM // 16)]
        dlo_vecs = [(16 * g + lane) & 7 for g in range(DIM // 16)]

        def gather_start(i, b):
            pltpu.make_async_copy(
                table_hbm.at[idx_v.at[pl.ds(i * _BB, _BB)]], gbuf[b],
                gsem[b]).start()

        def gather_wait(b):
            pltpu.make_async_copy(
                table_hbm.at[idx_v.at[pl.ds(0, _BB)]], gbuf[b],
                gsem[b]).wait()

        def out_descs(i, b):
            t = tbase + i
            l = t // NBB
            bh = lax.rem(t, NBB)
            return pltpu.make_async_copy(
                tbuf[b].at[:, :, pl.ds(0, _BB)], out_hbm.at[l, :, bh],
                osem[b])

        def out_start(i, b):
            out_descs(i, b).start()

        def out_wait(i, b):
            out_descs(i, b).wait()

        gather_start(0, 0)
        gather_start(1, 1)

        def step(cc, carry):
            for b in range(2):
                i = 2 * cc + b
                t = tbase + i
                l = t // NBB
                gather_wait(b)

                @pl.when(i >= 2)
                def _():
                    out_wait(i, b)

                gb, tb = gbuf[b], tbuf[b]
                pe_rows = [pe_v[l, pl.ds(16 * g, 16)]
                           for g in range(DIM // 16)]

                @plsc.parallel_loop(0, _BB, unroll=2)
                def _(r):
                    b_vec = jnp.full((16,), r, jnp.int32)
                    for g in range(DIM // 16):
                        v = gb[r, pl.ds(16 * g, 16)] + pe_rows[g]
                        plsc.store_scatter(
                            tb, [dh_vecs[g], dlo_vecs[g], b_vec], v)

                out_start(i, b)

                @pl.when(i + 2 < t_w)
                def _():
                    gather_start(i + 2, b)
            return carry

        lax.fori_loop(0, t_w // 2, step, 0)

        out_wait(t_w - 2, 0)
        out_wait(t_w - 1, 1)

    return k


def kernel(inputs, table):
    B, L = inputs.shape
    pe = jnp.asarray(_position_encoding(L, DIM))
    idx_t = inputs.T.reshape(-1)
    out5 = _make_kernel(B, L)(idx_t, table, pe)
    return out5.transpose((2, 4, 0, 1, 3)).reshape(B, L, DIM)
